# Initial kernel scaffold; baseline (speedup 1.0000x reference)
import jax
import jax.numpy as jnp
from jax.experimental import pallas as pl


def kernel(x, edge_index, time_features, W1, b1, W2, b2, Wt, bt, Wfc, bfc):
    raise NotImplementedError("write your pallas kernel here")


# speedup vs baseline: 23.5993x; 23.5993x over previous
import functools

import jax
import jax.numpy as jnp
from jax import lax
from jax.experimental import pallas as pl
from jax.experimental.pallas import tpu as pltpu
from jax.experimental.pallas import tpu_sc as plsc

N = 10000
E = 320000
D = 128
DH = D // 2

N_PAD = 10240
NC = 2
NS = 16
CHUNK = 128
E_PAD = 321536
NCH = E_PAD // (NS * CHUNK)
RPS = N_PAD // NS

_mesh = plsc.VectorSubcoreMesh(core_axis_name="c", subcore_axis_name="s")


DW = 16


def _deg_body(dst_hbm, zeros1_hbm, ones_hbm, out_hbm, dst_v, ones_v, acc_sh):
    cid = lax.axis_index("c")
    sid = lax.axis_index("s")
    pltpu.sync_copy(zeros1_hbm.at[pl.ds(sid * RPS, RPS)],
                    acc_sh.at[pl.ds(sid * RPS, RPS)])
    pltpu.sync_copy(ones_hbm, ones_v)
    pltpu.sync_copy(dst_hbm.at[sid], dst_v)
    plsc.subcore_barrier()

    def body(c, carry):
        pltpu.sync_copy(ones_v, acc_sh.at[dst_v.at[c]], add=True)
        return carry

    lax.fori_loop(0, NCH, body, 0)
    plsc.subcore_barrier()
    pltpu.sync_copy(acc_sh.at[pl.ds(sid * RPS, RPS)],
                    out_hbm.at[cid].at[pl.ds(sid * RPS, RPS)])


def _make_deg_kernel(interpret=False):
    return functools.partial(
        pl.kernel,
        out_type=jax.ShapeDtypeStruct((NC, N_PAD, DW), jnp.float32),
        mesh=_mesh,
        interpret=interpret,
        compiler_params=pltpu.CompilerParams(use_tc_tiling_on_sc=False),
        scratch_types=[
            pltpu.VMEM((NCH, CHUNK), jnp.int32),
            pltpu.VMEM((CHUNK, DW), jnp.float32),
            pltpu.VMEM_SHARED((N_PAD, DW), jnp.float32),
        ],
    )(_deg_body)


_deg_kernel = _make_deg_kernel()


def _agg_body(src_hbm, dst_hbm, u_hbm, zeros_hbm, out_hbm,
              src_v, dst_v, rows_v, acc_sh, gsem):
    cid = lax.axis_index("c")
    sid = lax.axis_index("s")
    pltpu.sync_copy(zeros_hbm.at[pl.ds(sid * RPS, RPS)],
                    acc_sh.at[pl.ds(sid * RPS, RPS)])
    pltpu.sync_copy(src_hbm.at[cid * NS + sid], src_v)
    pltpu.sync_copy(dst_hbm.at[sid], dst_v)
    plsc.subcore_barrier()

    pltpu.async_copy(u_hbm.at[src_v.at[0]], rows_v.at[0], gsem)

    def body(c, carry):
        @pl.when(c + 1 < NCH)
        def _():
            pltpu.async_copy(u_hbm.at[src_v.at[c + 1]],
                             rows_v.at[(c + 1) % 2], gsem)
        pltpu.make_async_copy(u_hbm.at[src_v.at[c]],
                              rows_v.at[c % 2], gsem).wait()
        pltpu.sync_copy(rows_v.at[c % 2], acc_sh.at[dst_v.at[c]], add=True)
        return carry

    lax.fori_loop(0, NCH, body, 0)
    plsc.subcore_barrier()
    pltpu.sync_copy(acc_sh.at[pl.ds(sid * RPS, RPS)],
                    out_hbm.at[cid].at[pl.ds(sid * RPS, RPS)])


def _make_agg_kernel(interpret=False):
    return functools.partial(
        pl.kernel,
        out_type=jax.ShapeDtypeStruct((NC, N_PAD, DH), jnp.float32),
        mesh=_mesh,
        interpret=interpret,
        compiler_params=pltpu.CompilerParams(use_tc_tiling_on_sc=False),
        scratch_types=[
            pltpu.VMEM((NCH, CHUNK), jnp.int32),
            pltpu.VMEM((NCH, CHUNK), jnp.int32),
            pltpu.VMEM((2, CHUNK, DH), jnp.float32),
            pltpu.VMEM_SHARED((N_PAD, DH), jnp.float32),
            pltpu.SemaphoreType.DMA,
        ],
    )(_agg_body)


_agg_kernel = _make_agg_kernel()


BR = 1280
GRID = N_PAD // BR


def _dinv_block(deg_blk):
    d = (deg_blk[0, :, 0:1] + deg_blk[1, :, 0:1]) * 0.5 + 1.0
    return lax.rsqrt(jnp.maximum(d, 1.0))


def _halves(agg_blk):
    return jnp.concatenate([agg_blk[0], agg_blk[1]], axis=1)


def _tc1_body(x_ref, w1_ref, deg_ref, u1_ref):
    dinv = _dinv_block(deg_ref[...])
    z = jnp.dot(x_ref[...], w1_ref[...], preferred_element_type=jnp.float32)
    u1_ref[...] = z * dinv


def _tc2_body(agg_ref, u1_ref, deg_ref, b1_ref, w2_ref, wfct_ref, u2_ref):
    dinv = _dinv_block(deg_ref[...])
    s = _halves(agg_ref[...]) + u1_ref[...]
    h1 = jnp.maximum(s * dinv + b1_ref[...], 0.0)
    w2p = jnp.dot(w2_ref[...], wfct_ref[...], preferred_element_type=jnp.float32)
    u2_ref[...] = jnp.dot(h1, w2p, preferred_element_type=jnp.float32) * dinv


def _tc3_body(agg_ref, u2_ref, deg_ref, tf_ref, wt_ref, bt_ref,
              wfcb_ref, b2_ref, wfct_ref, bfc_ref, out_ref):
    dinv = _dinv_block(deg_ref[...])
    g = (_halves(agg_ref[...]) + u2_ref[...]) * dinv
    tf = tf_ref[...]
    t = jnp.maximum(tf[:, 0:1] * wt_ref[0:1, :] + tf[:, 1:2] * wt_ref[1:2, :]
                    + bt_ref[...], 0.0)
    cvec = jnp.dot(b2_ref[...], wfct_ref[...],
                   preferred_element_type=jnp.float32) + bfc_ref[...]
    out_ref[...] = g + jnp.dot(t, wfcb_ref[...],
                               preferred_element_type=jnp.float32) + cvec


def _row_spec(width):
    return pl.BlockSpec((BR, width), lambda i: (i, 0))


_FULL = lambda shape: pl.BlockSpec(shape, lambda i: tuple(0 for _ in shape))
_DEG_SPEC = pl.BlockSpec((NC, BR, DW), lambda i: (0, i, 0))
_AGG_SPEC = pl.BlockSpec((NC, BR, DH), lambda i: (0, i, 0))


def kernel(x, edge_index, time_features, W1, b1, W2, b2, Wt, bt, Wfc, bfc):
    f32 = jnp.float32
    x = x.astype(f32)
    ei = edge_index.astype(jnp.int32)
    pad_ids = N + (jnp.arange(E_PAD - E, dtype=jnp.int32) % (N_PAD - N))
    src_flat = jnp.concatenate([ei[0], pad_ids])
    src_p = jnp.stack([src_flat, src_flat + N_PAD]).reshape(NC * NS, NCH, CHUNK)
    dst_p = jnp.concatenate([ei[1], pad_ids]).reshape(NS, NCH, CHUNK)
    x_p = jnp.pad(x, ((0, N_PAD - N), (0, 0)))
    tf_p = jnp.pad(time_features.astype(f32), ((0, N_PAD - N), (0, 0)))
    zeros_h = jnp.zeros((N_PAD, DH), f32)
    zeros1 = jnp.zeros((N_PAD, DW), f32)
    ones_c = jnp.ones((CHUNK, DW), f32)

    def stack_halves(u):
        return jnp.concatenate([u[:, :DH], u[:, DH:]], axis=0)

    degp = _deg_kernel(dst_p, zeros1, ones_c)

    tc1 = pl.pallas_call(
        _tc1_body,
        grid=(GRID,),
        in_specs=[_row_spec(D), _FULL((D, D)), _DEG_SPEC],
        out_specs=_row_spec(D),
        out_shape=jax.ShapeDtypeStruct((N_PAD, D), f32),
    )
    u1 = tc1(x_p, W1.astype(f32), degp)

    agg1 = _agg_kernel(src_p, dst_p, stack_halves(u1), zeros_h)

    tc2 = pl.pallas_call(
        _tc2_body,
        grid=(GRID,),
        in_specs=[_AGG_SPEC, _row_spec(D), _DEG_SPEC,
                  _FULL((1, D)), _FULL((D, D)), _FULL((D, D))],
        out_specs=_row_spec(D),
        out_shape=jax.ShapeDtypeStruct((N_PAD, D), f32),
    )
    u2 = tc2(agg1, u1, degp, b1.astype(f32).reshape(1, D),
             W2.astype(f32), Wfc[:D].astype(f32))

    agg2 = _agg_kernel(src_p, dst_p, stack_halves(u2), zeros_h)

    tc3 = pl.pallas_call(
        _tc3_body,
        grid=(GRID,),
        in_specs=[_AGG_SPEC, _row_spec(D), _DEG_SPEC, _row_spec(2),
                  _FULL((2, D)), _FULL((1, D)), _FULL((D, D)),
                  _FULL((1, D)), _FULL((D, D)), _FULL((1, D))],
        out_specs=_row_spec(D),
        out_shape=jax.ShapeDtypeStruct((N_PAD, D), f32),
    )
    out = tc3(agg2, u2, degp, tf_p, Wt.astype(f32),
              bt.astype(f32).reshape(1, D), Wfc[D:].astype(f32),
              b2.astype(f32).reshape(1, D), Wfc[:D].astype(f32),
              bfc.astype(f32).reshape(1, D))
    return out[:N]


# speedup vs baseline: 30.6367x; 1.2982x over previous
import functools

import jax
import jax.numpy as jnp
from jax import lax
from jax.experimental import pallas as pl
from jax.experimental.pallas import tpu as pltpu
from jax.experimental.pallas import tpu_sc as plsc

N = 10000
E = 320000
D = 128
DH = D // 2

N_PAD = 10240
NC = 2
NS = 16
CHUNK = 128
E_PAD = 321536
NCH = E_PAD // (NS * CHUNK)
RPS = N_PAD // NS
NBUF = 4
E_PADD = 323584
NCHD = E_PADD // (NC * NS * CHUNK)

_mesh = plsc.VectorSubcoreMesh(core_axis_name="c", subcore_axis_name="s")


DW = 16


def _deg_body(dst_hbm, zeros1_hbm, ones_hbm, out_hbm, dst_v, ones_v, acc_sh):
    cid = lax.axis_index("c")
    sid = lax.axis_index("s")
    pltpu.sync_copy(zeros1_hbm.at[pl.ds(sid * RPS, RPS)],
                    acc_sh.at[pl.ds(sid * RPS, RPS)])
    pltpu.sync_copy(ones_hbm, ones_v)
    pltpu.sync_copy(dst_hbm.at[cid * NS + sid], dst_v)
    plsc.subcore_barrier()

    def body(c, carry):
        pltpu.sync_copy(ones_v, acc_sh.at[dst_v.at[c]], add=True)
        return carry

    lax.fori_loop(0, NCHD, body, 0)
    plsc.subcore_barrier()
    pltpu.sync_copy(acc_sh.at[pl.ds(sid * RPS, RPS)],
                    out_hbm.at[cid].at[pl.ds(sid * RPS, RPS)])


def _make_deg_kernel(interpret=False):
    return functools.partial(
        pl.kernel,
        out_type=jax.ShapeDtypeStruct((NC, N_PAD, DW), jnp.float32),
        mesh=_mesh,
        interpret=interpret,
        compiler_params=pltpu.CompilerParams(use_tc_tiling_on_sc=False),
        scratch_types=[
            pltpu.VMEM((NCHD, CHUNK), jnp.int32),
            pltpu.VMEM((CHUNK, DW), jnp.float32),
            pltpu.VMEM_SHARED((N_PAD, DW), jnp.float32),
        ],
    )(_deg_body)


_deg_kernel = _make_deg_kernel()


def _agg_body(src_hbm, dst_hbm, u_hbm, zeros_hbm, out_hbm,
              src_v, dst_v, rows_v, acc_sh, gsem, ssem):
    cid = lax.axis_index("c")
    sid = lax.axis_index("s")
    pltpu.sync_copy(zeros_hbm.at[pl.ds(sid * RPS, RPS)],
                    acc_sh.at[pl.ds(sid * RPS, RPS)])
    pltpu.sync_copy(src_hbm.at[cid * NS + sid], src_v)
    pltpu.sync_copy(dst_hbm.at[sid], dst_v)
    plsc.subcore_barrier()

    pltpu.async_copy(u_hbm.at[src_v.at[0]], rows_v.at[0], gsem)
    pltpu.async_copy(u_hbm.at[src_v.at[1]], rows_v.at[1], gsem)

    def body(c, carry):
        @pl.when(c >= 2)
        def _():
            pltpu.make_async_copy(rows_v.at[(c - 2) % NBUF],
                                  acc_sh.at[dst_v.at[c - 2]], ssem).wait()
        pltpu.make_async_copy(u_hbm.at[src_v.at[c]],
                              rows_v.at[c % NBUF], gsem).wait()
        pltpu.async_copy(rows_v.at[c % NBUF],
                         acc_sh.at[dst_v.at[c]], ssem, add=True)

        @pl.when(c + 2 < NCH)
        def _():
            pltpu.async_copy(u_hbm.at[src_v.at[c + 2]],
                             rows_v.at[(c + 2) % NBUF], gsem)
        return carry

    lax.fori_loop(0, NCH, body, 0)
    pltpu.make_async_copy(rows_v.at[(NCH - 2) % NBUF],
                          acc_sh.at[dst_v.at[NCH - 2]], ssem).wait()
    pltpu.make_async_copy(rows_v.at[(NCH - 1) % NBUF],
                          acc_sh.at[dst_v.at[NCH - 1]], ssem).wait()
    plsc.subcore_barrier()
    pltpu.sync_copy(acc_sh.at[pl.ds(sid * RPS, RPS)],
                    out_hbm.at[cid].at[pl.ds(sid * RPS, RPS)])


def _make_agg_kernel(interpret=False):
    return functools.partial(
        pl.kernel,
        out_type=jax.ShapeDtypeStruct((NC, N_PAD, DH), jnp.float32),
        mesh=_mesh,
        interpret=interpret,
        compiler_params=pltpu.CompilerParams(use_tc_tiling_on_sc=False),
        scratch_types=[
            pltpu.VMEM((NCH, CHUNK), jnp.int32),
            pltpu.VMEM((NCH, CHUNK), jnp.int32),
            pltpu.VMEM((NBUF, CHUNK, DH), jnp.float32),
            pltpu.VMEM_SHARED((N_PAD, DH), jnp.float32),
            pltpu.SemaphoreType.DMA,
            pltpu.SemaphoreType.DMA,
        ],
    )(_agg_body)


_agg_kernel = _make_agg_kernel()


BR = 1280
GRID = N_PAD // BR


def _dinv_block(deg_blk):
    d = deg_blk[0, :, 0:1] + deg_blk[1, :, 0:1] + 1.0
    return lax.rsqrt(jnp.maximum(d, 1.0))


def _halves(agg_blk):
    return jnp.concatenate([agg_blk[0], agg_blk[1]], axis=1)


def _tc1_body(x_ref, w1_ref, deg_ref, u1_ref):
    dinv = _dinv_block(deg_ref[...])
    z = jnp.dot(x_ref[...], w1_ref[...], preferred_element_type=jnp.float32)
    u1_ref[...] = z * dinv


def _tc2_body(agg_ref, u1_ref, deg_ref, b1_ref, w2_ref, wfct_ref, u2_ref):
    dinv = _dinv_block(deg_ref[...])
    s = _halves(agg_ref[...]) + u1_ref[...]
    h1 = jnp.maximum(s * dinv + b1_ref[...], 0.0)
    w2p = jnp.dot(w2_ref[...], wfct_ref[...], preferred_element_type=jnp.float32)
    u2_ref[...] = jnp.dot(h1, w2p, preferred_element_type=jnp.float32) * dinv


def _tc3_body(agg_ref, u2_ref, deg_ref, tf_ref, wt_ref, bt_ref,
              wfcb_ref, b2_ref, wfct_ref, bfc_ref, out_ref):
    dinv = _dinv_block(deg_ref[...])
    g = (_halves(agg_ref[...]) + u2_ref[...]) * dinv
    tf = tf_ref[...]
    t = jnp.maximum(tf[:, 0:1] * wt_ref[0:1, :] + tf[:, 1:2] * wt_ref[1:2, :]
                    + bt_ref[...], 0.0)
    cvec = jnp.dot(b2_ref[...], wfct_ref[...],
                   preferred_element_type=jnp.float32) + bfc_ref[...]
    out_ref[...] = g + jnp.dot(t, wfcb_ref[...],
                               preferred_element_type=jnp.float32) + cvec


def _row_spec(width):
    return pl.BlockSpec((BR, width), lambda i: (i, 0))


_FULL = lambda shape: pl.BlockSpec(shape, lambda i: tuple(0 for _ in shape))
_DEG_SPEC = pl.BlockSpec((NC, BR, DW), lambda i: (0, i, 0))
_AGG_SPEC = pl.BlockSpec((NC, BR, DH), lambda i: (0, i, 0))


def kernel(x, edge_index, time_features, W1, b1, W2, b2, Wt, bt, Wfc, bfc):
    f32 = jnp.float32
    x = x.astype(f32)
    ei = edge_index.astype(jnp.int32)
    pad_ids = N + (jnp.arange(E_PAD - E, dtype=jnp.int32) % (N_PAD - N))
    src_flat = jnp.concatenate([ei[0], pad_ids])
    src_p = jnp.stack([2 * src_flat, 2 * src_flat + 1]).reshape(NC * NS, NCH, CHUNK)
    dst_p = jnp.concatenate([ei[1], pad_ids]).reshape(NS, NCH, CHUNK)
    pad_d = N + (jnp.arange(E_PADD - E, dtype=jnp.int32) % (N_PAD - N))
    dst_d = jnp.concatenate([ei[1], pad_d]).reshape(NC * NS, NCHD, CHUNK)
    x_p = jnp.pad(x, ((0, N_PAD - N), (0, 0)))
    tf_p = jnp.pad(time_features.astype(f32), ((0, N_PAD - N), (0, 0)))
    zeros_h = jnp.zeros((N_PAD, DH), f32)
    zeros1 = jnp.zeros((N_PAD, DW), f32)
    ones_c = jnp.ones((CHUNK, DW), f32)

    def stack_halves(u):
        return u.reshape(2 * N_PAD, DH)

    degp = _deg_kernel(dst_d, zeros1, ones_c)

    tc1 = pl.pallas_call(
        _tc1_body,
        grid=(GRID,),
        in_specs=[_row_spec(D), _FULL((D, D)), _DEG_SPEC],
        out_specs=_row_spec(D),
        out_shape=jax.ShapeDtypeStruct((N_PAD, D), f32),
    )
    u1 = tc1(x_p, W1.astype(f32), degp)

    agg1 = _agg_kernel(src_p, dst_p, stack_halves(u1), zeros_h)

    tc2 = pl.pallas_call(
        _tc2_body,
        grid=(GRID,),
        in_specs=[_AGG_SPEC, _row_spec(D), _DEG_SPEC,
                  _FULL((1, D)), _FULL((D, D)), _FULL((D, D))],
        out_specs=_row_spec(D),
        out_shape=jax.ShapeDtypeStruct((N_PAD, D), f32),
    )
    u2 = tc2(agg1, u1, degp, b1.astype(f32).reshape(1, D),
             W2.astype(f32), Wfc[:D].astype(f32))

    agg2 = _agg_kernel(src_p, dst_p, stack_halves(u2), zeros_h)

    tc3 = pl.pallas_call(
        _tc3_body,
        grid=(GRID,),
        in_specs=[_AGG_SPEC, _row_spec(D), _DEG_SPEC, _row_spec(2),
                  _FULL((2, D)), _FULL((1, D)), _FULL((D, D)),
                  _FULL((1, D)), _FULL((D, D)), _FULL((1, D))],
        out_specs=_row_spec(D),
        out_shape=jax.ShapeDtypeStruct((N_PAD, D), f32),
    )
    out = tc3(agg2, u2, degp, tf_p, Wt.astype(f32),
              bt.astype(f32).reshape(1, D), Wfc[D:].astype(f32),
              b2.astype(f32).reshape(1, D), Wfc[:D].astype(f32),
              bfc.astype(f32).reshape(1, D))
    return out[:N]


# speedup vs baseline: 33.0803x; 1.0798x over previous
import functools

import jax
import jax.numpy as jnp
from jax import lax
from jax.experimental import pallas as pl
from jax.experimental.pallas import tpu as pltpu
from jax.experimental.pallas import tpu_sc as plsc

N = 10000
E = 320000
D = 128
DH = D // 2

N_PAD = 10240
NC = 2
NS = 16
CHUNK = 128
E_PAD = 321536
NCH = E_PAD // (NS * CHUNK)
RPS = N_PAD // NS
NBUF = 6
GLEAD = 3
E_PADD = 323584
NCHD = E_PADD // (NC * NS * CHUNK)

_mesh = plsc.VectorSubcoreMesh(core_axis_name="c", subcore_axis_name="s")


DW = 16


def _deg_body(dst_hbm, zeros1_hbm, ones_hbm, out_hbm, dst_v, ones_v, acc_sh):
    cid = lax.axis_index("c")
    sid = lax.axis_index("s")
    pltpu.sync_copy(zeros1_hbm.at[pl.ds(sid * RPS, RPS)],
                    acc_sh.at[pl.ds(sid * RPS, RPS)])
    pltpu.sync_copy(ones_hbm, ones_v)
    pltpu.sync_copy(dst_hbm.at[cid * NS + sid], dst_v)
    plsc.subcore_barrier()

    def body(c, carry):
        pltpu.sync_copy(ones_v, acc_sh.at[dst_v.at[c]], add=True)
        return carry

    lax.fori_loop(0, NCHD, body, 0)
    plsc.subcore_barrier()
    pltpu.sync_copy(acc_sh.at[pl.ds(sid * RPS, RPS)],
                    out_hbm.at[cid].at[pl.ds(sid * RPS, RPS)])


def _make_deg_kernel(interpret=False):
    return functools.partial(
        pl.kernel,
        out_type=jax.ShapeDtypeStruct((NC, N_PAD, DW), jnp.float32),
        mesh=_mesh,
        interpret=interpret,
        compiler_params=pltpu.CompilerParams(use_tc_tiling_on_sc=False),
        scratch_types=[
            pltpu.VMEM((NCHD, CHUNK), jnp.int32),
            pltpu.VMEM((CHUNK, DW), jnp.float32),
            pltpu.VMEM_SHARED((N_PAD, DW), jnp.float32),
        ],
    )(_deg_body)


_deg_kernel = _make_deg_kernel()


def _agg_body(src_hbm, dst_hbm, u_hbm, zeros_hbm, out_hbm,
              src_v, dst_v, rows_v, acc_sh, gsem, ssem):
    cid = lax.axis_index("c")
    sid = lax.axis_index("s")
    pltpu.sync_copy(zeros_hbm.at[pl.ds(sid * RPS, RPS)],
                    acc_sh.at[pl.ds(sid * RPS, RPS)])
    pltpu.sync_copy(src_hbm.at[cid * NS + sid], src_v)
    pltpu.sync_copy(dst_hbm.at[sid], dst_v)
    plsc.subcore_barrier()

    for c0 in range(GLEAD):
        pltpu.async_copy(u_hbm.at[src_v.at[c0]], rows_v.at[c0], gsem)

    def body(c, carry):
        @pl.when(c >= GLEAD)
        def _():
            pltpu.make_async_copy(rows_v.at[(c - GLEAD) % NBUF],
                                  acc_sh.at[dst_v.at[c - GLEAD]], ssem).wait()
        pltpu.make_async_copy(u_hbm.at[src_v.at[c]],
                              rows_v.at[c % NBUF], gsem).wait()
        pltpu.async_copy(rows_v.at[c % NBUF],
                         acc_sh.at[dst_v.at[c]], ssem, add=True)

        @pl.when(c + GLEAD < NCH)
        def _():
            pltpu.async_copy(u_hbm.at[src_v.at[c + GLEAD]],
                             rows_v.at[(c + GLEAD) % NBUF], gsem)
        return carry

    lax.fori_loop(0, NCH, body, 0)
    for c0 in range(NCH - GLEAD, NCH):
        pltpu.make_async_copy(rows_v.at[c0 % NBUF],
                              acc_sh.at[dst_v.at[c0]], ssem).wait()
    plsc.subcore_barrier()
    pltpu.sync_copy(acc_sh.at[pl.ds(sid * RPS, RPS)],
                    out_hbm.at[cid].at[pl.ds(sid * RPS, RPS)])


def _make_agg_kernel(interpret=False):
    return functools.partial(
        pl.kernel,
        out_type=jax.ShapeDtypeStruct((NC, N_PAD, DH), jnp.float32),
        mesh=_mesh,
        interpret=interpret,
        compiler_params=pltpu.CompilerParams(use_tc_tiling_on_sc=False),
        scratch_types=[
            pltpu.VMEM((NCH, CHUNK), jnp.int32),
            pltpu.VMEM((NCH, CHUNK), jnp.int32),
            pltpu.VMEM((NBUF, CHUNK, DH), jnp.float32),
            pltpu.VMEM_SHARED((N_PAD, DH), jnp.float32),
            pltpu.SemaphoreType.DMA,
            pltpu.SemaphoreType.DMA,
        ],
    )(_agg_body)


_agg_kernel = _make_agg_kernel()


BR = 1280
GRID = N_PAD // BR


def _dinv_block(deg_blk):
    d = deg_blk[0, :, 0:1] + deg_blk[1, :, 0:1] + 1.0
    return lax.rsqrt(jnp.maximum(d, 1.0))


def _halves(agg_blk):
    return jnp.concatenate([agg_blk[0], agg_blk[1]], axis=1)


def _tc1_body(x_ref, w1_ref, deg_ref, u1_ref):
    dinv = _dinv_block(deg_ref[...])
    z = jnp.dot(x_ref[...], w1_ref[...], preferred_element_type=jnp.float32)
    u1_ref[...] = z * dinv


def _tc2_body(agg_ref, u1_ref, deg_ref, b1_ref, w2_ref, wfct_ref, u2_ref):
    dinv = _dinv_block(deg_ref[...])
    s = _halves(agg_ref[...]) + u1_ref[...]
    h1 = jnp.maximum(s * dinv + b1_ref[...], 0.0)
    w2p = jnp.dot(w2_ref[...], wfct_ref[...], preferred_element_type=jnp.float32)
    u2_ref[...] = jnp.dot(h1, w2p, preferred_element_type=jnp.float32) * dinv


def _tc3_body(agg_ref, u2_ref, deg_ref, tf_ref, wt_ref, bt_ref,
              wfcb_ref, b2_ref, wfct_ref, bfc_ref, out_ref):
    dinv = _dinv_block(deg_ref[...])
    g = (_halves(agg_ref[...]) + u2_ref[...]) * dinv
    tf = tf_ref[...]
    t = jnp.maximum(tf[:, 0:1] * wt_ref[0:1, :] + tf[:, 1:2] * wt_ref[1:2, :]
                    + bt_ref[...], 0.0)
    cvec = jnp.dot(b2_ref[...], wfct_ref[...],
                   preferred_element_type=jnp.float32) + bfc_ref[...]
    out_ref[...] = g + jnp.dot(t, wfcb_ref[...],
                               preferred_element_type=jnp.float32) + cvec


def _row_spec(width):
    return pl.BlockSpec((BR, width), lambda i: (i, 0))


_FULL = lambda shape: pl.BlockSpec(shape, lambda i: tuple(0 for _ in shape))
_DEG_SPEC = pl.BlockSpec((NC, BR, DW), lambda i: (0, i, 0))
_AGG_SPEC = pl.BlockSpec((NC, BR, DH), lambda i: (0, i, 0))


def kernel(x, edge_index, time_features, W1, b1, W2, b2, Wt, bt, Wfc, bfc):
    f32 = jnp.float32
    x = x.astype(f32)
    ei = edge_index.astype(jnp.int32)
    pad_ids = N + (jnp.arange(E_PAD - E, dtype=jnp.int32) % (N_PAD - N))
    src_flat = jnp.concatenate([ei[0], pad_ids])
    src_p = jnp.stack([2 * src_flat, 2 * src_flat + 1]).reshape(NC * NS, NCH, CHUNK)
    dst_p = jnp.concatenate([ei[1], pad_ids]).reshape(NS, NCH, CHUNK)
    pad_d = N + (jnp.arange(E_PADD - E, dtype=jnp.int32) % (N_PAD - N))
    dst_d = jnp.concatenate([ei[1], pad_d]).reshape(NC * NS, NCHD, CHUNK)
    x_p = jnp.pad(x, ((0, N_PAD - N), (0, 0)))
    tf_p = jnp.pad(time_features.astype(f32), ((0, N_PAD - N), (0, 0)))
    zeros_h = jnp.zeros((N_PAD, DH), f32)
    zeros1 = jnp.zeros((N_PAD, DW), f32)
    ones_c = jnp.ones((CHUNK, DW), f32)

    def stack_halves(u):
        return u.reshape(2 * N_PAD, DH)

    degp = _deg_kernel(dst_d, zeros1, ones_c)

    tc1 = pl.pallas_call(
        _tc1_body,
        grid=(GRID,),
        in_specs=[_row_spec(D), _FULL((D, D)), _DEG_SPEC],
        out_specs=_row_spec(D),
        out_shape=jax.ShapeDtypeStruct((N_PAD, D), f32),
    )
    u1 = tc1(x_p, W1.astype(f32), degp)

    agg1 = _agg_kernel(src_p, dst_p, stack_halves(u1), zeros_h)

    tc2 = pl.pallas_call(
        _tc2_body,
        grid=(GRID,),
        in_specs=[_AGG_SPEC, _row_spec(D), _DEG_SPEC,
                  _FULL((1, D)), _FULL((D, D)), _FULL((D, D))],
        out_specs=_row_spec(D),
        out_shape=jax.ShapeDtypeStruct((N_PAD, D), f32),
    )
    u2 = tc2(agg1, u1, degp, b1.astype(f32).reshape(1, D),
             W2.astype(f32), Wfc[:D].astype(f32))

    agg2 = _agg_kernel(src_p, dst_p, stack_halves(u2), zeros_h)

    tc3 = pl.pallas_call(
        _tc3_body,
        grid=(GRID,),
        in_specs=[_AGG_SPEC, _row_spec(D), _DEG_SPEC, _row_spec(2),
                  _FULL((2, D)), _FULL((1, D)), _FULL((D, D)),
                  _FULL((1, D)), _FULL((D, D)), _FULL((1, D))],
        out_specs=_row_spec(D),
        out_shape=jax.ShapeDtypeStruct((N_PAD, D), f32),
    )
    out = tc3(agg2, u2, degp, tf_p, Wt.astype(f32),
              bt.astype(f32).reshape(1, D), Wfc[D:].astype(f32),
              b2.astype(f32).reshape(1, D), Wfc[:D].astype(f32),
              bfc.astype(f32).reshape(1, D))
    return out[:N]


# speedup vs baseline: 33.2313x; 1.0046x over previous
import functools

import jax
import jax.numpy as jnp
from jax import lax
from jax.experimental import pallas as pl
from jax.experimental.pallas import tpu as pltpu
from jax.experimental.pallas import tpu_sc as plsc

N = 10000
E = 320000
D = 128
DH = D // 2

N_PAD = 10240
NC = 2
NS = 16
CHUNK = 128
E_PAD = 321536
NCH = E_PAD // (NS * CHUNK)
RPS = N_PAD // NS
NBUF = 6
GLEAD = 3
E_PADD = 323584
NCHD = E_PADD // (NC * NS * CHUNK)

_mesh = plsc.VectorSubcoreMesh(core_axis_name="c", subcore_axis_name="s")


DW = 16


def _deg_body(dst_hbm, zeros1_hbm, ones_hbm, out_hbm, dst_v, ones_v, acc_sh):
    cid = lax.axis_index("c")
    sid = lax.axis_index("s")
    pltpu.sync_copy(zeros1_hbm.at[pl.ds(sid * RPS, RPS)],
                    acc_sh.at[pl.ds(sid * RPS, RPS)])
    pltpu.sync_copy(ones_hbm, ones_v)
    pltpu.sync_copy(dst_hbm.at[cid * NS + sid], dst_v)
    plsc.subcore_barrier()

    def body(c, carry):
        pltpu.sync_copy(ones_v, acc_sh.at[dst_v.at[c]], add=True)
        return carry

    lax.fori_loop(0, NCHD, body, 0)
    plsc.subcore_barrier()
    pltpu.sync_copy(acc_sh.at[pl.ds(sid * RPS, RPS)],
                    out_hbm.at[cid].at[pl.ds(sid * RPS, RPS)])


def _make_deg_kernel(interpret=False):
    return functools.partial(
        pl.kernel,
        out_type=jax.ShapeDtypeStruct((NC, N_PAD, DW), jnp.float32),
        mesh=_mesh,
        interpret=interpret,
        compiler_params=pltpu.CompilerParams(use_tc_tiling_on_sc=False),
        scratch_types=[
            pltpu.VMEM((NCHD, CHUNK), jnp.int32),
            pltpu.VMEM((CHUNK, DW), jnp.float32),
            pltpu.VMEM_SHARED((N_PAD, DW), jnp.float32),
        ],
    )(_deg_body)


_deg_kernel = _make_deg_kernel()


def _agg_body(src_hbm, dst_hbm, u_hbm, zeros_hbm, out_hbm,
              src_v, dst_v, rows_v, acc_sh, gsem, ssem):
    cid = lax.axis_index("c")
    sid = lax.axis_index("s")
    pltpu.sync_copy(zeros_hbm.at[pl.ds(sid * RPS, RPS)],
                    acc_sh.at[pl.ds(sid * RPS, RPS)])
    pltpu.sync_copy(src_hbm.at[cid * NS + sid], src_v)
    pltpu.sync_copy(dst_hbm.at[sid], dst_v)
    plsc.subcore_barrier()

    for c0 in range(GLEAD):
        pltpu.async_copy(u_hbm.at[src_v.at[c0]], rows_v.at[c0], gsem)

    def body(c, carry):
        @pl.when(c >= GLEAD)
        def _():
            pltpu.make_async_copy(rows_v.at[(c - GLEAD) % NBUF],
                                  acc_sh.at[dst_v.at[c - GLEAD]], ssem).wait()
        pltpu.make_async_copy(u_hbm.at[src_v.at[c]],
                              rows_v.at[c % NBUF], gsem).wait()
        pltpu.async_copy(rows_v.at[c % NBUF],
                         acc_sh.at[dst_v.at[c]], ssem, add=True)

        @pl.when(c + GLEAD < NCH)
        def _():
            pltpu.async_copy(u_hbm.at[src_v.at[c + GLEAD]],
                             rows_v.at[(c + GLEAD) % NBUF], gsem)
        return carry

    lax.fori_loop(0, NCH, body, 0)
    for c0 in range(NCH - GLEAD, NCH):
        pltpu.make_async_copy(rows_v.at[c0 % NBUF],
                              acc_sh.at[dst_v.at[c0]], ssem).wait()
    plsc.subcore_barrier()
    pltpu.sync_copy(acc_sh.at[pl.ds(sid * RPS, RPS)],
                    out_hbm.at[cid].at[pl.ds(sid * RPS, RPS)])


def _make_agg_kernel(interpret=False):
    return functools.partial(
        pl.kernel,
        out_type=jax.ShapeDtypeStruct((NC, N_PAD, DH), jnp.float32),
        mesh=_mesh,
        interpret=interpret,
        compiler_params=pltpu.CompilerParams(use_tc_tiling_on_sc=False),
        scratch_types=[
            pltpu.VMEM((NCH, CHUNK), jnp.int32),
            pltpu.VMEM((NCH, CHUNK), jnp.int32),
            pltpu.VMEM((NBUF, CHUNK, DH), jnp.float32),
            pltpu.VMEM_SHARED((N_PAD, DH), jnp.float32),
            pltpu.SemaphoreType.DMA,
            pltpu.SemaphoreType.DMA,
        ],
    )(_agg_body)


_agg_kernel = _make_agg_kernel()


BR = 1280
GRID = N_PAD // BR


def _dinv_block(deg_blk):
    d = deg_blk[0, :, 0:1] + deg_blk[1, :, 0:1] + 1.0
    return lax.rsqrt(jnp.maximum(d, 1.0))


def _halves(agg_blk):
    return jnp.concatenate([agg_blk[0], agg_blk[1]], axis=1)


def _tc1a_body(x_ref, w1_ref, z1_ref):
    z1_ref[...] = jnp.dot(x_ref[...], w1_ref[...],
                          preferred_element_type=jnp.float32)


def _tc1b_body(z1_ref, deg_ref, u1_ref):
    u1_ref[...] = z1_ref[...] * _dinv_block(deg_ref[...])


def _tct_body(tf_ref, wt_ref, bt_ref, wfcb_ref, b2_ref, wfct_ref, bfc_ref,
              tpart_ref):
    tf = tf_ref[...]
    t = jnp.maximum(tf[:, 0:1] * wt_ref[0:1, :] + tf[:, 1:2] * wt_ref[1:2, :]
                    + bt_ref[...], 0.0)
    cvec = jnp.dot(b2_ref[...], wfct_ref[...],
                   preferred_element_type=jnp.float32) + bfc_ref[...]
    tpart_ref[...] = jnp.dot(t, wfcb_ref[...],
                             preferred_element_type=jnp.float32) + cvec


def _tc2_body(agg_ref, u1_ref, deg_ref, b1_ref, w2_ref, wfct_ref, u2_ref):
    dinv = _dinv_block(deg_ref[...])
    s = _halves(agg_ref[...]) + u1_ref[...]
    h1 = jnp.maximum(s * dinv + b1_ref[...], 0.0)
    w2p = jnp.dot(w2_ref[...], wfct_ref[...], preferred_element_type=jnp.float32)
    u2_ref[...] = jnp.dot(h1, w2p, preferred_element_type=jnp.float32) * dinv


def _tc3_body(agg_ref, u2_ref, deg_ref, tpart_ref, out_ref):
    dinv = _dinv_block(deg_ref[...])
    out_ref[...] = (_halves(agg_ref[...]) + u2_ref[...]) * dinv + tpart_ref[...]


def _row_spec(width, rows=BR):
    return pl.BlockSpec((rows, width), lambda i: (i, 0))


_FULL = lambda shape: pl.BlockSpec(shape, lambda i: tuple(0 for _ in shape))
_DEG_SPEC = pl.BlockSpec((NC, BR, DW), lambda i: (0, i, 0))
_AGG_SPEC = pl.BlockSpec((NC, BR, DH), lambda i: (0, i, 0))
BR3 = 1000
GRID3 = N // BR3
_DEG3 = pl.BlockSpec((NC, BR3, DW), lambda i: (0, i, 0))
_AGG3 = pl.BlockSpec((NC, BR3, DH), lambda i: (0, i, 0))


def kernel(x, edge_index, time_features, W1, b1, W2, b2, Wt, bt, Wfc, bfc):
    f32 = jnp.float32
    x = x.astype(f32)
    ei = edge_index.astype(jnp.int32)
    pad_ids = N + (jnp.arange(E_PAD - E, dtype=jnp.int32) % (N_PAD - N))
    src_flat = jnp.concatenate([ei[0], pad_ids])
    src_p = jnp.stack([2 * src_flat, 2 * src_flat + 1]).reshape(NC * NS, NCH, CHUNK)
    dst_p = jnp.concatenate([ei[1], pad_ids]).reshape(NS, NCH, CHUNK)
    pad_d = N + (jnp.arange(E_PADD - E, dtype=jnp.int32) % (N_PAD - N))
    dst_d = jnp.concatenate([ei[1], pad_d]).reshape(NC * NS, NCHD, CHUNK)
    x_p = jnp.pad(x, ((0, N_PAD - N), (0, 0)))
    tf_p = jnp.pad(time_features.astype(f32), ((0, N_PAD - N), (0, 0)))
    zeros_h = jnp.zeros((N_PAD, DH), f32)
    zeros1 = jnp.zeros((N_PAD, DW), f32)
    ones_c = jnp.ones((CHUNK, DW), f32)

    def stack_halves(u):
        return u.reshape(2 * N_PAD, DH)

    degp = _deg_kernel(dst_d, zeros1, ones_c)

    tc1a = pl.pallas_call(
        _tc1a_body,
        grid=(GRID,),
        in_specs=[_row_spec(D), _FULL((D, D))],
        out_specs=_row_spec(D),
        out_shape=jax.ShapeDtypeStruct((N_PAD, D), f32),
    )
    z1 = tc1a(x_p, W1.astype(f32))

    tc1b = pl.pallas_call(
        _tc1b_body,
        grid=(GRID,),
        in_specs=[_row_spec(D), _DEG_SPEC],
        out_specs=_row_spec(D),
        out_shape=jax.ShapeDtypeStruct((N_PAD, D), f32),
    )
    u1 = tc1b(z1, degp)

    tct = pl.pallas_call(
        _tct_body,
        grid=(GRID,),
        in_specs=[_row_spec(2), _FULL((2, D)), _FULL((1, D)), _FULL((D, D)),
                  _FULL((1, D)), _FULL((D, D)), _FULL((1, D))],
        out_specs=_row_spec(D),
        out_shape=jax.ShapeDtypeStruct((N_PAD, D), f32),
    )
    tpart = tct(tf_p, Wt.astype(f32), bt.astype(f32).reshape(1, D),
                Wfc[D:].astype(f32), b2.astype(f32).reshape(1, D),
                Wfc[:D].astype(f32), bfc.astype(f32).reshape(1, D))

    agg1 = _agg_kernel(src_p, dst_p, stack_halves(u1), zeros_h)

    tc2 = pl.pallas_call(
        _tc2_body,
        grid=(GRID,),
        in_specs=[_AGG_SPEC, _row_spec(D), _DEG_SPEC,
                  _FULL((1, D)), _FULL((D, D)), _FULL((D, D))],
        out_specs=_row_spec(D),
        out_shape=jax.ShapeDtypeStruct((N_PAD, D), f32),
    )
    u2 = tc2(agg1, u1, degp, b1.astype(f32).reshape(1, D),
             W2.astype(f32), Wfc[:D].astype(f32))

    agg2 = _agg_kernel(src_p, dst_p, stack_halves(u2), zeros_h)

    tc3 = pl.pallas_call(
        _tc3_body,
        grid=(GRID3,),
        in_specs=[_AGG3, _row_spec(D, BR3), _DEG3, _row_spec(D, BR3)],
        out_specs=_row_spec(D, BR3),
        out_shape=jax.ShapeDtypeStruct((N, D), f32),
    )
    return tc3(agg2, u2, degp, tpart)
